# baseline (speedup 1.0000x reference)
import functools

import jax
import jax.numpy as jnp
from jax import lax
from jax.experimental import pallas as pl
from jax.experimental.pallas import tpu as pltpu
from jax.experimental.pallas import tpu_sc as plsc

B = 2
CIN = 64
D = 16
S = 512
COUT = CIN + D
RCH = 32
RR = S // RCH
NCH = 4 * RCH
KSLOT = 4

_mesh = plsc.VectorSubcoreMesh(core_axis_name="c", subcore_axis_name="s")


@functools.partial(
    pl.kernel,
    mesh=_mesh,
    out_type=jax.ShapeDtypeStruct((B, COUT, S, S), jnp.float32),
    scratch_types=[
        pltpu.VMEM((KSLOT, RR, S), jnp.float32),
        pltpu.VMEM((S, D), jnp.float32),
        pltpu.VMEM((2, 16, S), jnp.float32),
        pltpu.SemaphoreType.DMA,
        pltpu.SemaphoreType.DMA,
        pltpu.SemaphoreType.DMA,
        pltpu.SemaphoreType.DMA,
    ],
)
def _concat_dist_emb(inp, table, out, buf, e_v, w_v, sem_in, sem_out,
                     sem_tab, sem_win):
    w = lax.axis_index("s") * 2 + lax.axis_index("c")

    tab_dma = pltpu.async_copy(table, e_v, sem_tab)

    b_cp = w // 16
    ch0 = (w % 16) * 4

    def copy_refs(n):
        chan = ch0 + n // RCH
        r0 = pl.multiple_of((n % RCH) * RR, RR)
        slot = lax.rem(n, KSLOT)
        return inp.at[b_cp, chan, pl.ds(r0, RR)], buf.at[slot], \
            out.at[b_cp, chan, pl.ds(r0, RR)]

    def fire_gather(n):
        src, stage, _ = copy_refs(n)
        pltpu.async_copy(src, stage, sem_in)

    def wait_gather(n):
        src, stage, _ = copy_refs(n)
        pltpu.make_async_copy(src, stage, sem_in).wait()

    def fire_scatter(n):
        _, stage, dst = copy_refs(n)
        pltpu.async_copy(stage, dst, sem_out)

    def wait_scatter(n):
        _, stage, dst = copy_refs(n)
        pltpu.make_async_copy(stage, dst, sem_out).wait()

    ch = CIN + w // 2
    row_base = 256 * (w % 2)

    def win_refs(t, b):
        slot = lax.rem(t, 2)
        r0 = pl.multiple_of(row_base + 16 * t, 16)
        return w_v.at[slot], out.at[b, ch, pl.ds(r0, 16)]

    def assemble(t):
        i = 16 * w + t
        slot = lax.rem(t, 2)

        def body(r, carry):
            m0 = 32 * r - i
            for c in range(32):
                w_v[slot, r, pl.ds(16 * c, 16)] = e_v[jnp.abs(m0 + c)]
            return carry

        lax.fori_loop(0, 16, body, 0)

    tab_dma.wait()
    for p in range(KSLOT - 1):
        fire_gather(p)

    def loop_body(n, carry):
        @pl.when(n + KSLOT - 1 < NCH)
        def _():
            @pl.when(n >= 1)
            def _():
                wait_scatter(n - 1)
            fire_gather(n + KSLOT - 1)

        wait_gather(n)
        fire_scatter(n)

        @pl.when(lax.rem(n, NCH // 16) == NCH // 16 - 1)
        def _():
            t = n // (NCH // 16)

            @pl.when(t >= 2)
            def _():
                for b in range(B):
                    src, dst = win_refs(t - 2, b)
                    pltpu.make_async_copy(src, dst, sem_win).wait()

            assemble(t)
            for b in range(B):
                src, dst = win_refs(t, b)
                pltpu.async_copy(src, dst, sem_win)

        return carry

    lax.fori_loop(0, NCH, loop_body, 0)

    for p in range(KSLOT):
        wait_scatter(NCH - KSLOT + p)
    for t in (14, 15):
        for b in range(B):
            src, dst = win_refs(t, b)
            pltpu.make_async_copy(src, dst, sem_win).wait()


def kernel(inputs, emb_table):
    return _concat_dist_emb(inputs, emb_table)


# speedup vs baseline: 1.0036x; 1.0036x over previous
import functools

import jax
import jax.numpy as jnp
from jax import lax
from jax.experimental import pallas as pl
from jax.experimental.pallas import tpu as pltpu
from jax.experimental.pallas import tpu_sc as plsc

B = 2
CIN = 64
D = 16
S = 512
COUT = CIN + D
RCH = 16
RR = S // RCH
NCH = 4 * RCH

_mesh = plsc.VectorSubcoreMesh(core_axis_name="c", subcore_axis_name="s")


@functools.partial(
    pl.kernel,
    mesh=_mesh,
    out_type=jax.ShapeDtypeStruct((B, COUT, S, S), jnp.float32),
    scratch_types=[
        pltpu.VMEM((2, RR, S), jnp.float32),
        pltpu.VMEM((S, D), jnp.float32),
        pltpu.VMEM((2, 16, S), jnp.float32),
        pltpu.SemaphoreType.DMA,
        pltpu.SemaphoreType.DMA,
        pltpu.SemaphoreType.DMA,
        pltpu.SemaphoreType.DMA,
    ],
)
def _concat_dist_emb(inp, table, out, buf, e_v, w_v, sem_in, sem_out,
                     sem_tab, sem_win):
    w = lax.axis_index("s") * 2 + lax.axis_index("c")

    tab_dma = pltpu.async_copy(table, e_v, sem_tab)

    b_cp = w // 16
    ch0 = (w % 16) * 4

    def copy_refs(n):
        chan = ch0 + n // RCH
        r0 = pl.multiple_of((n % RCH) * RR, RR)
        slot = lax.rem(n, 2)
        return inp.at[b_cp, chan, pl.ds(r0, RR)], buf.at[slot], \
            out.at[b_cp, chan, pl.ds(r0, RR)]

    def fire_gather(n):
        src, stage, _ = copy_refs(n)
        pltpu.async_copy(src, stage, sem_in)

    def wait_gather(n):
        src, stage, _ = copy_refs(n)
        pltpu.make_async_copy(src, stage, sem_in).wait()

    def fire_scatter(n):
        _, stage, dst = copy_refs(n)
        pltpu.async_copy(stage, dst, sem_out)

    def wait_scatter(n):
        _, stage, dst = copy_refs(n)
        pltpu.make_async_copy(stage, dst, sem_out).wait()

    ch = CIN + w // 2
    row_base = 256 * (w % 2)

    def win_refs(t, b):
        slot = lax.rem(t, 2)
        r0 = pl.multiple_of(row_base + 16 * t, 16)
        return w_v.at[slot], out.at[b, ch, pl.ds(r0, 16)]

    def assemble(t):
        i = 16 * w + t
        slot = lax.rem(t, 2)

        def body(r, carry):
            m0 = 32 * r - i
            for c in range(32):
                w_v[slot, r, pl.ds(16 * c, 16)] = e_v[jnp.abs(m0 + c)]
            return carry

        lax.fori_loop(0, 16, body, 0)

    tab_dma.wait()
    fire_gather(0)

    def loop_body(n, carry):
        @pl.when(n + 1 < NCH)
        def _():
            @pl.when(n >= 1)
            def _():
                wait_scatter(n - 1)
            fire_gather(n + 1)

        wait_gather(n)
        fire_scatter(n)

        @pl.when(lax.rem(n, 4) == 3)
        def _():
            t = n // 4

            @pl.when(t >= 2)
            def _():
                for b in range(B):
                    src, dst = win_refs(t - 2, b)
                    pltpu.make_async_copy(src, dst, sem_win).wait()

            assemble(t)
            for b in range(B):
                src, dst = win_refs(t, b)
                pltpu.async_copy(src, dst, sem_win)

        return carry

    lax.fori_loop(0, NCH, loop_body, 0)

    wait_scatter(NCH - 2)
    wait_scatter(NCH - 1)
    for t in (14, 15):
        for b in range(B):
            src, dst = win_refs(t, b)
            pltpu.make_async_copy(src, dst, sem_win).wait()


def kernel(inputs, emb_table):
    return _concat_dist_emb(inputs, emb_table)
